# baseline (speedup 1.0000x reference)
import jax
import jax.numpy as jnp
from jax.experimental import pallas as pl
from jax.experimental.pallas import tpu as pltpu

CHUNK = 500
NBUF = 4
ROWS = 4 * 10000
NCHUNK = ROWS // CHUNK


def _kern(x_hbm, emb_hbm, o_hbm, xbuf, ebuf, obuf, in_sem, emb_sem, out_sem):
    def in_copy(c, slot):
        row0 = c * CHUNK
        v0 = row0 % 10000
        return (
            pltpu.make_async_copy(
                x_hbm.at[pl.ds(row0, CHUNK)], xbuf.at[slot], in_sem.at[slot]
            ),
            pltpu.make_async_copy(
                emb_hbm.at[pl.ds(v0, CHUNK)], ebuf.at[slot], emb_sem.at[slot]
            ),
        )

    def out_copy(c, slot):
        row0 = c * CHUNK
        return pltpu.make_async_copy(
            obuf.at[slot], o_hbm.at[pl.ds(row0, CHUNK)], out_sem.at[slot]
        )

    for s in range(min(NBUF, NCHUNK)):
        a, b = in_copy(s, s)
        a.start()
        b.start()

    for c in range(NCHUNK):
        slot = c % NBUF
        a, b = in_copy(c, slot)
        a.wait()
        b.wait()
        if c >= NBUF:
            out_copy(c - NBUF, slot).wait()
        obuf[slot] = xbuf[slot] + ebuf[slot][:, None, :]
        out_copy(c, slot).start()
        nc = c + NBUF
        if nc < NCHUNK:
            a, b = in_copy(nc, slot)
            a.start()
            b.start()

    for c in range(max(NCHUNK - NBUF, 0), NCHUNK):
        out_copy(c, c % NBUF).wait()


def kernel(x, emb_weight):
    batch, n, t, d = x.shape
    x2 = x.reshape(batch * n, t, d)
    out = pl.pallas_call(
        _kern,
        in_specs=[
            pl.BlockSpec(memory_space=pl.ANY),
            pl.BlockSpec(memory_space=pl.ANY),
        ],
        out_specs=pl.BlockSpec(memory_space=pl.ANY),
        out_shape=jax.ShapeDtypeStruct((batch * n, t, d), x.dtype),
        scratch_shapes=[
            pltpu.VMEM((NBUF, CHUNK, t, d), x.dtype),
            pltpu.VMEM((NBUF, CHUNK, d), x.dtype),
            pltpu.VMEM((NBUF, CHUNK, t, d), x.dtype),
            pltpu.SemaphoreType.DMA((NBUF,)),
            pltpu.SemaphoreType.DMA((NBUF,)),
            pltpu.SemaphoreType.DMA((NBUF,)),
        ],
    )(x2, emb_weight)
    return out.reshape(batch, n, t, d)
